# baseline (speedup 1.0000x reference)
import jax
import jax.numpy as jnp
from jax import lax
from jax.experimental import pallas as pl
from jax.experimental.pallas import tpu as pltpu
from jax.experimental.pallas import tpu_sc as plsc

V = 1_000_000
K = 32
B = 16384
F = 26
L = 16

NC = 2
NS = 16
NW = NC * NS
IPW = B // NW
CHUNK = 64
NCH = IPW // CHUNK
IDXC = CHUNK * F


def _fm_body(emb_hbm, xf_hbm, biasf_hbm, w0_hbm, out_hbm,
             idx0, idx1, rows0, rows1, bias0, bias1, logit0, logit1,
             w0_v, gsem):
    wid = lax.axis_index("s") * NC + lax.axis_index("c")
    pltpu.sync_copy(w0_hbm, w0_v.at[pl.ds(0, 1)])
    w0s = w0_v[pl.ds(0, L)][0]

    idxb = (idx0, idx1)
    rowsb = (rows0, rows1)
    biasb = (bias0, bias1)
    logitb = (logit0, logit1)

    lane = lax.iota(jnp.int32, L)
    tail_mask = lane < (F - L)

    def issue(c):
        p = c & 1
        xoff = wid * (IPW * F) + c * IDXC
        pltpu.sync_copy(xf_hbm.at[pl.ds(xoff, IDXC)], idxb[p])
        e = pltpu.async_copy(emb_hbm.at[idxb[p]], rowsb[p], gsem.at[p])
        bcp = pltpu.async_copy(
            biasf_hbm.at[idxb[p]], biasb[p].at[pl.ds(0, IDXC)], gsem.at[p])
        return [e, bcp]

    def compute(c):
        p = c & 1
        rows_v = rowsb[p]
        bias_v = biasb[p]
        logit_v = logitb[p]

        def item(i, lacc):
            base = i * F
            s0 = jnp.zeros((L,), jnp.float32)
            s1 = jnp.zeros((L,), jnp.float32)
            q0 = jnp.zeros((L,), jnp.float32)
            q1 = jnp.zeros((L,), jnp.float32)
            for f in range(F):
                r0 = rows_v[base + f, pl.ds(0, L)]
                r1 = rows_v[base + f, pl.ds(L, L)]
                s0 = s0 + r0
                s1 = s1 + r1
                q0 = q0 + r0 * r0
                q1 = q1 + r1 * r1
            acc = s0 * s0 + s1 * s1 - q0 - q1
            b0 = bias_v[pl.ds(base, L)]
            b1 = jnp.where(tail_mask, bias_v[pl.ds(base + L, L)], 0.0)
            t = 0.5 * acc + b0 + b1
            lacc = jnp.where(lane == lax.rem(i, L), jnp.sum(t), lacc)

            @pl.when(lax.rem(i, L) == L - 1)
            def _():
                logit_v[pl.ds(i - (L - 1), L)] = lacc

            return lacc

        lax.fori_loop(0, CHUNK, item, jnp.zeros((L,), jnp.float32))

        for j in range(CHUNK // L):
            x = logit_v[pl.ds(j * L, L)]
            y = 5.5 / (1.0 + jnp.exp(-(x + w0s)))
            logit_v[pl.ds(j * L, L)] = y
        pltpu.sync_copy(logit_v,
                        out_hbm.at[pl.ds(wid * IPW + c * CHUNK, CHUNK)])

    pend = issue(0)
    for c in range(NCH):
        nxt = issue(c + 1) if c + 1 < NCH else None
        for cp in pend:
            cp.wait()
        compute(c)
        pend = nxt


@jax.jit
def _fm_call(xf, emb_table, biasf, w0):
    mesh = plsc.VectorSubcoreMesh(core_axis_name="c", subcore_axis_name="s")
    fn = pl.kernel(
        _fm_body,
        out_type=jax.ShapeDtypeStruct((B,), jnp.float32),
        mesh=mesh,
        scratch_types=[
            pltpu.VMEM((IDXC,), jnp.int32),
            pltpu.VMEM((IDXC,), jnp.int32),
            pltpu.VMEM((IDXC, K), jnp.float32),
            pltpu.VMEM((IDXC, K), jnp.float32),
            pltpu.VMEM((IDXC + L,), jnp.float32),
            pltpu.VMEM((IDXC + L,), jnp.float32),
            pltpu.VMEM((CHUNK,), jnp.float32),
            pltpu.VMEM((CHUNK,), jnp.float32),
            pltpu.VMEM((L,), jnp.float32),
            pltpu.SemaphoreType.DMA((2,)),
        ],
        compiler_params=pltpu.CompilerParams(
            needs_layout_passes=False, use_tc_tiling_on_sc=False),
    )
    return fn(emb_table, xf, biasf, w0)


def kernel(X, emb_table, bias_table, w0):
    xf = X.reshape(-1).astype(jnp.int32)
    biasf = bias_table.reshape(-1)
    return _fm_call(xf, emb_table, biasf, w0)
